# baseline (speedup 1.0000x reference)
import functools

import jax
import jax.numpy as jnp
from jax import lax
from jax.experimental import pallas as pl
from jax.experimental.pallas import tpu as pltpu
from jax.experimental.pallas import tpu_sc as plsc

N = 10000
D = 128
E = 320000
EPS = 1e-5

NC = 2
NS = 16
NW = NC * NS
EPT = E // NW
BLK = 80
NB = EPT // BLK
NPAD = 10240
DWPT = NPAD // NS
RPT = NPAD // NS
ZROWS = 128

_mesh = plsc.VectorSubcoreMesh(core_axis_name="c", subcore_axis_name="s")


@functools.partial(
    pl.kernel,
    out_type=jax.ShapeDtypeStruct((NC, NPAD), jnp.float32),
    mesh=_mesh,
    scratch_types=[
        pltpu.VMEM((DWPT,), jnp.float32),
        pltpu.VMEM((NB, BLK), jnp.int32),
        pltpu.VMEM((NB, BLK), jnp.float32),
        pltpu.VMEM_SHARED((NPAD,), jnp.float32),
    ],
)
def _deg_call(dst_hbm, ew_hbm, out_hbm, zb, dsti, ewi, accd):
    c = lax.axis_index("c")
    s = lax.axis_index("s")
    wid = c * NS + s

    pltpu.sync_copy(dst_hbm.at[wid], dsti)
    pltpu.sync_copy(ew_hbm.at[wid], ewi)

    @pl.loop(0, DWPT // 16)
    def _zero(i):
        zb[pl.ds(i * 16, 16)] = jnp.zeros((16,), jnp.float32)

    pltpu.sync_copy(zb, accd.at[pl.ds(s * DWPT, DWPT)])
    plsc.subcore_barrier()

    @pl.loop(0, NB)
    def _blocks(i):
        pltpu.sync_copy(ewi.at[i], accd.at[dsti.at[i]], add=True)

    plsc.subcore_barrier()
    pltpu.sync_copy(accd.at[pl.ds(s * DWPT, DWPT)],
                    out_hbm.at[c, pl.ds(s * DWPT, DWPT)])


@functools.partial(
    pl.kernel,
    out_type=jax.ShapeDtypeStruct((NC, NPAD, D), jnp.float32),
    mesh=_mesh,
    scratch_types=[
        pltpu.VMEM((EPT,), jnp.int32),
        pltpu.VMEM((EPT,), jnp.int32),
        pltpu.VMEM((BLK,), jnp.int32),
        pltpu.VMEM((BLK,), jnp.int32),
        pltpu.VMEM((BLK,), jnp.float32),
        pltpu.VMEM((BLK,), jnp.float32),
        pltpu.VMEM((BLK, D), jnp.float32),
        pltpu.VMEM((BLK, D), jnp.float32),
        pltpu.VMEM_SHARED((NPAD, D), jnp.float32),
        pltpu.SemaphoreType.DMA,
        pltpu.SemaphoreType.DMA,
        pltpu.SemaphoreType.DMA,
        pltpu.SemaphoreType.DMA,
    ],
)
def _edge_call(g_hbm, src_hbm, dst_hbm, ew_hbm, out_hbm,
               srcf, dstf, dstv0, dstv1, ewv0, ewv1, rows0, rows1,
               acc, sem0, sem1, ssem0, ssem1):
    c = lax.axis_index("c")
    s = lax.axis_index("s")
    wid = c * NS + s

    pltpu.sync_copy(src_hbm.at[pl.ds(wid * EPT, EPT)], srcf)
    pltpu.sync_copy(dst_hbm.at[pl.ds(wid * EPT, EPT)], dstf)

    @pl.loop(0, BLK)
    def _zero(r):
        for j in range(D // 16):
            rows0[r, pl.ds(j * 16, 16)] = jnp.zeros((16,), jnp.float32)

    for k in range(RPT // BLK):
        pltpu.sync_copy(rows0, acc.at[pl.ds(s * RPT + k * BLK, BLK)])
    plsc.subcore_barrier()

    def _start_blk(i, dstv, ewv, rows, sem):
        for k in range(BLK // 16):
            dstv[pl.ds(k * 16, 16)] = dstf[pl.ds(i * BLK + k * 16, 16)]
        pltpu.async_copy(ew_hbm.at[pl.ds(wid * EPT + i * BLK, BLK)], ewv, sem)
        pltpu.async_copy(g_hbm.at[srcf.at[pl.ds(i * BLK, BLK)]], rows, sem)

    def _wait_blk(i, ewv, rows, sem):
        pltpu.make_async_copy(
            ew_hbm.at[pl.ds(wid * EPT + i * BLK, BLK)], ewv, sem).wait()
        pltpu.make_async_copy(
            g_hbm.at[srcf.at[pl.ds(i * BLK, BLK)]], rows, sem).wait()

    def _scale(ewv, buf):
        @pl.loop(0, BLK // 16)
        def _s(b16):
            wv = ewv[pl.ds(b16 * 16, 16)]
            for k in range(16):
                spl = wv[k]
                b = b16 * 16 + k
                for j in range(D // 16):
                    sl = pl.ds(j * 16, 16)
                    buf[b, sl] = buf[b, sl] * spl

    def _start_scatter(dstv, buf, ssem):
        pltpu.async_copy(buf, acc.at[dstv], ssem, add=True)

    def _wait_scatter(dstv, buf, ssem):
        pltpu.make_async_copy(buf, acc.at[dstv], ssem).wait()

    _start_blk(0, dstv0, ewv0, rows0, sem0)

    @pl.loop(0, NB // 2)
    def _blocks(k):
        i = 2 * k

        @pl.when(k > 0)
        def _():
            _wait_scatter(dstv1, rows1, ssem1)

        _start_blk(i + 1, dstv1, ewv1, rows1, sem1)
        _wait_blk(i, ewv0, rows0, sem0)
        _scale(ewv0, rows0)
        _start_scatter(dstv0, rows0, ssem0)
        _wait_blk(i + 1, ewv1, rows1, sem1)
        _scale(ewv1, rows1)
        _wait_scatter(dstv0, rows0, ssem0)
        _start_blk(i + 2, dstv0, ewv0, rows0, sem0)
        _start_scatter(dstv1, rows1, ssem1)

    _wait_scatter(dstv1, rows1, ssem1)
    _wait_blk(NB - 1, ewv0, rows0, sem0)
    _scale(ewv0, rows0)
    _start_scatter(dstv0, rows0, ssem0)
    _wait_scatter(dstv0, rows0, ssem0)

    plsc.subcore_barrier()
    for k in range(RPT // ZROWS):
        r0 = s * RPT + k * ZROWS
        pltpu.sync_copy(acc.at[pl.ds(r0, ZROWS)], out_hbm.at[c, pl.ds(r0, ZROWS)])


def _dense1_body(x_ref, w_ref, deg_ref, h1_ref, g1_ref):
    h1 = jnp.dot(x_ref[...], w_ref[...], preferred_element_type=jnp.float32)
    dinv = lax.rsqrt(deg_ref[...] + 1.0)
    h1_ref[...] = h1
    g1_ref[...] = h1 * dinv


def _dense1(x, W1, deg_col):
    return pl.pallas_call(
        _dense1_body,
        out_shape=(jax.ShapeDtypeStruct((N, D), jnp.float32),
                   jax.ShapeDtypeStruct((N, D), jnp.float32)),
    )(x, W1, deg_col)


def _dense2_body(s_ref, h1_ref, deg_ref, gamma_ref, beta_ref, b1_ref,
                 w2_ref, h2_ref, g2_ref):
    dinv = lax.rsqrt(deg_ref[...] + 1.0)
    h = dinv * (s_ref[0, :N] + s_ref[1, :N]) + (dinv * dinv) * h1_ref[...] \
        + b1_ref[...]
    mean = jnp.mean(h, axis=0, keepdims=True)
    hm = h - mean
    var = jnp.mean(hm * hm, axis=0, keepdims=True)
    hn = hm * lax.rsqrt(var + EPS) * gamma_ref[...] + beta_ref[...]
    r = jnp.maximum(hn, 0.0)
    h2 = jnp.dot(r, w2_ref[...], preferred_element_type=jnp.float32)
    h2_ref[...] = h2
    g2_ref[...] = h2 * dinv


def _dense2(s, h1, deg_col, gamma, beta, b1, W2):
    return pl.pallas_call(
        _dense2_body,
        out_shape=(jax.ShapeDtypeStruct((N, D), jnp.float32),
                   jax.ShapeDtypeStruct((N, D), jnp.float32)),
    )(s, h1, deg_col, gamma, beta, b1, W2)


def _dense3_body(s_ref, h2_ref, deg_ref, b2_ref, out_ref):
    dinv = lax.rsqrt(deg_ref[...] + 1.0)
    out_ref[...] = dinv * (s_ref[0, :N] + s_ref[1, :N]) \
        + (dinv * dinv) * h2_ref[...] + b2_ref[...]


def _dense3(s, h2, deg_col, b2):
    return pl.pallas_call(
        _dense3_body,
        out_shape=jax.ShapeDtypeStruct((N, D), jnp.float32),
    )(s, h2, deg_col, b2)


def kernel(t, x_nodes, edge_index, edge_weight, W1, b1, gamma1, beta1, W2, b2):
    src = edge_index[0]
    dst = edge_index[1]
    dst3 = dst.reshape(NW, NB, BLK)
    ew3 = edge_weight.reshape(NW, NB, BLK)

    degp = _deg_call(dst3, ew3)
    deg_col = (degp[0, :N] + degp[1, :N]).reshape(N, 1)

    h1, g1 = _dense1(x_nodes, W1, deg_col)
    s1 = _edge_call(g1, src, dst, edge_weight)
    h2, g2 = _dense2(s1, h1, deg_col,
                     gamma1.reshape(1, D), beta1.reshape(1, D),
                     b1.reshape(1, D), W2)
    s2 = _edge_call(g2, src, dst, edge_weight)
    dz = _dense3(s2, h2, deg_col, b2.reshape(1, D))
    return dz


# speedup vs baseline: 1.1414x; 1.1414x over previous
import functools

import jax
import jax.numpy as jnp
from jax import lax
from jax.experimental import pallas as pl
from jax.experimental.pallas import tpu as pltpu
from jax.experimental.pallas import tpu_sc as plsc

N = 10000
D = 128
E = 320000
EPS = 1e-5

NC = 2
NS = 16
NW = NC * NS
EPT = E // NW
BLK = 80
NB = EPT // BLK
EB = 64
ENB = EPT // EB
TAIL = EPT - ENB * EB
Q = 4
NPAD = 10240
DWPT = NPAD // NS
RPT = NPAD // NS
ZROWS = 128

_mesh = plsc.VectorSubcoreMesh(core_axis_name="c", subcore_axis_name="s")


@functools.partial(
    pl.kernel,
    out_type=jax.ShapeDtypeStruct((NC, NPAD), jnp.float32),
    mesh=_mesh,
    scratch_types=[
        pltpu.VMEM((DWPT,), jnp.float32),
        pltpu.VMEM((NB, BLK), jnp.int32),
        pltpu.VMEM((NB, BLK), jnp.float32),
        pltpu.VMEM_SHARED((NPAD,), jnp.float32),
    ],
)
def _deg_call(dst_hbm, ew_hbm, out_hbm, zb, dsti, ewi, accd):
    c = lax.axis_index("c")
    s = lax.axis_index("s")
    wid = c * NS + s

    pltpu.sync_copy(dst_hbm.at[wid], dsti)
    pltpu.sync_copy(ew_hbm.at[wid], ewi)

    @pl.loop(0, DWPT // 16)
    def _zero(i):
        zb[pl.ds(i * 16, 16)] = jnp.zeros((16,), jnp.float32)

    pltpu.sync_copy(zb, accd.at[pl.ds(s * DWPT, DWPT)])
    plsc.subcore_barrier()

    @pl.loop(0, NB)
    def _blocks(i):
        pltpu.sync_copy(ewi.at[i], accd.at[dsti.at[i]], add=True)

    plsc.subcore_barrier()
    pltpu.sync_copy(accd.at[pl.ds(s * DWPT, DWPT)],
                    out_hbm.at[c, pl.ds(s * DWPT, DWPT)])


@functools.partial(
    pl.kernel,
    out_type=jax.ShapeDtypeStruct((NC, NPAD, D), jnp.float32),
    mesh=_mesh,
    scratch_types=[
        pltpu.VMEM((EPT,), jnp.int32),
        [pltpu.VMEM((EB,), jnp.int32) for _ in range(Q)],
        [pltpu.VMEM((EB,), jnp.float32) for _ in range(Q)],
        [pltpu.VMEM((EB, D), jnp.float32) for _ in range(Q)],
        pltpu.VMEM((TAIL,), jnp.int32),
        pltpu.VMEM((TAIL,), jnp.float32),
        pltpu.VMEM((TAIL, D), jnp.float32),
        pltpu.VMEM_SHARED((NPAD, D), jnp.float32),
        [pltpu.SemaphoreType.DMA for _ in range(Q)],
        [pltpu.SemaphoreType.DMA for _ in range(Q)],
        pltpu.SemaphoreType.DMA,
    ],
)
def _edge_call(g_hbm, src_hbm, dst_hbm, ew_hbm, out_hbm,
               srcf, dstv, ewv, rows, dstt, ewt, rowst,
               acc, gsem, ssem, tsem):
    c = lax.axis_index("c")
    s = lax.axis_index("s")
    wid = c * NS + s

    pltpu.sync_copy(src_hbm.at[pl.ds(wid * EPT, EPT)], srcf)

    @pl.loop(0, EB)
    def _zero(r):
        for j in range(D // 16):
            rows[0][r, pl.ds(j * 16, 16)] = jnp.zeros((16,), jnp.float32)

    for k in range(RPT // EB):
        pltpu.sync_copy(rows[0], acc.at[pl.ds(s * RPT + k * EB, EB)])
    plsc.subcore_barrier()

    def _start_blk(i, j):
        off = wid * EPT + i * EB
        pltpu.async_copy(dst_hbm.at[pl.ds(off, EB)], dstv[j], gsem[j])
        pltpu.async_copy(ew_hbm.at[pl.ds(off, EB)], ewv[j], gsem[j])
        pltpu.async_copy(g_hbm.at[srcf.at[pl.ds(i * EB, EB)]], rows[j], gsem[j])

    def _wait_blk(i, j):
        off = wid * EPT + i * EB
        pltpu.make_async_copy(dst_hbm.at[pl.ds(off, EB)], dstv[j], gsem[j]).wait()
        pltpu.make_async_copy(ew_hbm.at[pl.ds(off, EB)], ewv[j], gsem[j]).wait()
        pltpu.make_async_copy(
            g_hbm.at[srcf.at[pl.ds(i * EB, EB)]], rows[j], gsem[j]).wait()

    def _scale(ew_b, buf, nb):
        @plsc.parallel_loop(0, nb // 16)
        def _s(b16):
            wv = ew_b[pl.ds(b16 * 16, 16)]
            for k in range(16):
                spl = wv[k]
                b = b16 * 16 + k
                for j in range(D // 16):
                    sl = pl.ds(j * 16, 16)
                    buf[b, sl] = buf[b, sl] * spl

    for j in range(Q - 1):
        _start_blk(j, j)

    @pl.loop(0, ENB // Q)
    def _blocks(kk):
        for j in range(Q):
            i = Q * kk + j

            _wait_blk(i, j)
            _scale(ewv[j], rows[j], EB)
            pltpu.async_copy(rows[j], acc.at[dstv[j]], ssem[j], add=True)

            jn = (j + Q - 1) % Q

            @pl.when(i >= 1)
            def _():
                pltpu.make_async_copy(rows[jn], acc.at[dstv[jn]], ssem[jn]).wait()

            @pl.when(i + Q - 1 < ENB)
            def _():
                _start_blk(i + Q - 1, jn)

    jl = (ENB - 1) % Q
    pltpu.make_async_copy(rows[jl], acc.at[dstv[jl]], ssem[jl]).wait()

    offt = wid * EPT + ENB * EB
    pltpu.async_copy(dst_hbm.at[pl.ds(offt, TAIL)], dstt, tsem)
    pltpu.async_copy(ew_hbm.at[pl.ds(offt, TAIL)], ewt, tsem)
    pltpu.async_copy(g_hbm.at[srcf.at[pl.ds(ENB * EB, TAIL)]], rowst, tsem)
    pltpu.make_async_copy(dst_hbm.at[pl.ds(offt, TAIL)], dstt, tsem).wait()
    pltpu.make_async_copy(ew_hbm.at[pl.ds(offt, TAIL)], ewt, tsem).wait()
    pltpu.make_async_copy(
        g_hbm.at[srcf.at[pl.ds(ENB * EB, TAIL)]], rowst, tsem).wait()
    _scale(ewt, rowst, TAIL)
    pltpu.sync_copy(rowst, acc.at[dstt], add=True)

    plsc.subcore_barrier()
    for k in range(RPT // ZROWS):
        r0 = s * RPT + k * ZROWS
        pltpu.sync_copy(acc.at[pl.ds(r0, ZROWS)], out_hbm.at[c, pl.ds(r0, ZROWS)])


def _dense1_body(x_ref, w_ref, deg_ref, h1_ref, g1_ref):
    h1 = jnp.dot(x_ref[...], w_ref[...], preferred_element_type=jnp.float32)
    dinv = lax.rsqrt(deg_ref[...] + 1.0)
    h1_ref[...] = h1
    g1_ref[...] = h1 * dinv


def _dense1(x, W1, deg_col):
    return pl.pallas_call(
        _dense1_body,
        out_shape=(jax.ShapeDtypeStruct((N, D), jnp.float32),
                   jax.ShapeDtypeStruct((N, D), jnp.float32)),
    )(x, W1, deg_col)


def _dense2_body(s_ref, h1_ref, deg_ref, gamma_ref, beta_ref, b1_ref,
                 w2_ref, h2_ref, g2_ref):
    dinv = lax.rsqrt(deg_ref[...] + 1.0)
    h = dinv * (s_ref[0, :N] + s_ref[1, :N]) + (dinv * dinv) * h1_ref[...] \
        + b1_ref[...]
    mean = jnp.mean(h, axis=0, keepdims=True)
    hm = h - mean
    var = jnp.mean(hm * hm, axis=0, keepdims=True)
    hn = hm * lax.rsqrt(var + EPS) * gamma_ref[...] + beta_ref[...]
    r = jnp.maximum(hn, 0.0)
    h2 = jnp.dot(r, w2_ref[...], preferred_element_type=jnp.float32)
    h2_ref[...] = h2
    g2_ref[...] = h2 * dinv


def _dense2(s, h1, deg_col, gamma, beta, b1, W2):
    return pl.pallas_call(
        _dense2_body,
        out_shape=(jax.ShapeDtypeStruct((N, D), jnp.float32),
                   jax.ShapeDtypeStruct((N, D), jnp.float32)),
    )(s, h1, deg_col, gamma, beta, b1, W2)


def _dense3_body(s_ref, h2_ref, deg_ref, b2_ref, out_ref):
    dinv = lax.rsqrt(deg_ref[...] + 1.0)
    out_ref[...] = dinv * (s_ref[0, :N] + s_ref[1, :N]) \
        + (dinv * dinv) * h2_ref[...] + b2_ref[...]


def _dense3(s, h2, deg_col, b2):
    return pl.pallas_call(
        _dense3_body,
        out_shape=jax.ShapeDtypeStruct((N, D), jnp.float32),
    )(s, h2, deg_col, b2)


def kernel(t, x_nodes, edge_index, edge_weight, W1, b1, gamma1, beta1, W2, b2):
    src = edge_index[0]
    dst = edge_index[1]
    dst3 = dst.reshape(NW, NB, BLK)
    ew3 = edge_weight.reshape(NW, NB, BLK)

    degp = _deg_call(dst3, ew3)
    deg_col = (degp[0, :N] + degp[1, :N]).reshape(N, 1)

    h1, g1 = _dense1(x_nodes, W1, deg_col)
    s1 = _edge_call(g1, src, dst, edge_weight)
    h2, g2 = _dense2(s1, h1, deg_col,
                     gamma1.reshape(1, D), beta1.reshape(1, D),
                     b1.reshape(1, D), W2)
    s2 = _edge_call(g2, src, dst, edge_weight)
    dz = _dense3(s2, h2, deg_col, b2.reshape(1, D))
    return dz
